# baseline (speedup 1.0000x reference)
import functools

import jax
import jax.numpy as jnp
from jax import lax
from jax.experimental import pallas as pl
from jax.experimental.pallas import tpu as pltpu
from jax.experimental.pallas import tpu_sc as plsc

F32 = jnp.float32
I32 = jnp.int32

N = 10000
NP = 10240
NG = 64
NCLS = 7
ETOT = 170000
EP = 172032
EPT = EP // 32
B = 192
NBLK = EPT // B
EPS = EP // 16
B2 = 512
NBLK2 = EPS // B2
RPT = NP // 16
NEG = -1.0e30

@functools.cache
def _mesh():
    return plsc.VectorSubcoreMesh(core_axis_name="c", subcore_axis_name="s",
                                  num_cores=2, num_subcores=16)


def _leaky(z):
    return jnp.where(z > 0, z, z * 0.2)


def _make_e1(H, h, NH):
    def body(xl_hbm, xr_hbm, att_hbm, s_hbm, d_hbm, e_hbm, mx_hbm,
             sfull, dfull, il, ir, xlr, xrr, efull, attv, mxv, gsl, gsr):
        cid = lax.axis_index("c")
        sid = lax.axis_index("s")
        wid = sid * 2 + cid
        base = wid * EPT

        pltpu.sync_copy(att_hbm.at[h], attv)
        attc = [attv[pl.ds(k * 16, 16)] for k in range(8)]
        mxv[...] = jnp.full((16,), NEG, F32)
        pltpu.sync_copy(s_hbm.at[pl.ds(base, EPT)], sfull)
        pltpu.sync_copy(d_hbm.at[pl.ds(base, EPT)], dfull)
        i0 = lax.iota(I32, 16)
        shuf = [(i0 ^ 8).reshape(16, 1), (i0 ^ 4).reshape(16, 1),
                (i0 ^ 2).reshape(16, 1), (i0 ^ 1).reshape(16, 1)]
        _dn = lax.GatherDimensionNumbers(offset_dims=(),
                                         collapsed_slice_dims=(0,),
                                         start_index_map=(0,))

        def _perm(v, ix):
            return lax.gather(v, ix, _dn, (1,),
                              mode=lax.GatherScatterMode.PROMISE_IN_BOUNDS)

        def stage(i, b):
            off = i * B
            for g in range(B // 16):
                sv = sfull[pl.ds(off + g * 16, 16)]
                dv = dfull[pl.ds(off + g * 16, 16)]
                if H == 1:
                    il[b][pl.ds(g * 16, 16)] = sv
                    ir[b][pl.ds(g * 16, 16)] = dv
                else:
                    il[b][pl.ds(g * 16, 16)] = sv * H + h
                    ir[b][pl.ds(g * 16, 16)] = dv * H + h
            pltpu.async_copy(xl_hbm.at[il[b]], xlr[b], gsl[b])
            pltpu.async_copy(xr_hbm.at[ir[b]], xrr[b], gsr[b])

        def waitg(b):
            pltpu.make_async_copy(xl_hbm.at[pl.ds(0, B)], xlr[b], gsl[b]).wait()
            pltpu.make_async_copy(xr_hbm.at[pl.ds(0, B)], xrr[b], gsr[b]).wait()

        def compute(i, b):
            eb = base + i * B
            off = i * B

            def edge(j, carry):
                acc = jnp.zeros((16,), F32)
                for k in range(8):
                    z = xlr[b][j, pl.ds(k * 16, 16)] + xrr[b][j, pl.ds(k * 16, 16)]
                    acc = acc + _leaky(z) * attc[k]
                for ix in shuf:
                    acc = acc + _perm(acc, ix)
                plsc.store_scatter(efull, [jnp.full((16,), off + j, I32)], acc)
                return carry

            lax.fori_loop(0, B, edge, 0)
            for g in range(B // 16):
                ev = efull[pl.ds(off + g * 16, 16)]
                gid = eb + g * 16 + lax.iota(I32, 16)
                ev = jnp.where(gid < ETOT, ev, NEG)
                efull[pl.ds(off + g * 16, 16)] = ev
                mxv[...] = jnp.maximum(mxv[...], ev)

        stage(0, 0)

        def pair(g, carry):
            for b in (0, 1):
                i = g * 2 + b

                @pl.when(i + 1 < NBLK)
                def _():
                    stage(i + 1, 1 - b)

                waitg(b)
                compute(i, b)
            return carry

        lax.fori_loop(0, NBLK // 2, pair, 0)
        pltpu.sync_copy(efull, e_hbm.at[pl.ds(base, EPT)])
        pltpu.sync_copy(mxv, mx_hbm.at[wid])

    scratch = dict(
        sfull=pltpu.VMEM((EPT,), I32),
        dfull=pltpu.VMEM((EPT,), I32),
        il=[pltpu.VMEM((B,), I32)] * 2,
        ir=[pltpu.VMEM((B,), I32)] * 2,
        xlr=[pltpu.VMEM((B, 128), F32)] * 2,
        xrr=[pltpu.VMEM((B, 128), F32)] * 2,
        efull=pltpu.VMEM((EPT,), F32),
        attv=pltpu.VMEM((128,), F32),
        mxv=pltpu.VMEM((16,), F32),
        gsl=[pltpu.SemaphoreType.DMA] * 2,
        gsr=[pltpu.SemaphoreType.DMA] * 2,
    )
    return pl.kernel(
        body,
        out_type=(jax.ShapeDtypeStruct((EP,), F32),
                  jax.ShapeDtypeStruct((32, 16), F32)),
        mesh=_mesh(),
        scratch_types=scratch,
        compiler_params=pltpu.CompilerParams(needs_layout_passes=False, use_tc_tiling_on_sc=False),
    )


BA = 32
NBLKA = EPT // BA
SB = 8
SBE = SB * BA
NSUP = NBLKA // SB
B2 = 64
B2S = 512
NS2 = EPS // B2S


def _make_e3(H, h, NH):
    def body(xl_hbm, s_hbm, d_hbm, e_hbm, mx_hbm, part_hbm,
             sfe, dfe, efe, db, il, abuf, eb2, db2, dbs, exr2, es, xlr, mbuf,
             dencol, dentile, den_sh, den1d_sh, out_sh, gs, ss):
        cid = lax.axis_index("c")
        sid = lax.axis_index("s")
        r0 = sid * RPT

        def zex(j, carry):
            exr2[0][j, pl.ds(0, 16)] = jnp.zeros((16,), F32)
            exr2[1][j, pl.ds(0, 16)] = jnp.zeros((16,), F32)
            return carry

        lax.fori_loop(0, B2, zex, 0)

        def zxl(j, carry):
            for k in range(8):
                xlr[0][j, pl.ds(k * 16, 16)] = jnp.zeros((16,), F32)
            return carry

        lax.fori_loop(0, BA, zxl, 0)
        for q in range(RPT // BA):
            pltpu.sync_copy(xlr[0], out_sh.at[pl.ds(r0 + q * BA, BA)])
        for q in range(RPT // B2):
            pltpu.sync_copy(exr2[0], den_sh.at[pl.ds(r0 + q * B2, B2)])
        plsc.subcore_barrier()

        pltpu.sync_copy(mx_hbm, mbuf)
        mv = mbuf[0, pl.ds(0, 16)]
        for i in range(1, 32):
            mv = jnp.maximum(mv, mbuf[i, pl.ds(0, 16)])
        K = jnp.max(mv)

        def dsuper(q, carry):
            off = sid * EPS + q * B2S
            pltpu.sync_copy(e_hbm.at[pl.ds(off, B2S)], eb2)
            pltpu.sync_copy(d_hbm.at[pl.ds(off, B2S)], db2)
            for tt in range(B2S // B2):
                b = tt % 2
                if tt >= 2:
                    pltpu.make_async_copy(exr2[b], den_sh.at[pl.ds(0, B2)],
                                          es[b]).wait()
                for g in range(B2 // 16):
                    ev = eb2[pl.ds(tt * B2 + g * 16, 16)]
                    ex = jnp.exp(ev - K)
                    rows = lax.iota(I32, 16) + g * 16
                    plsc.store_scatter(exr2[b], [rows, jnp.zeros((16,), I32)],
                                       ex)
                    dbs[b][pl.ds(g * 16, 16)] = db2[pl.ds(tt * B2 + g * 16, 16)]
                pltpu.async_copy(exr2[b], den_sh.at[dbs[b]], es[b], add=True)
            for b in (0, 1):
                pltpu.make_async_copy(exr2[b], den_sh.at[pl.ds(0, B2)],
                                      es[b]).wait()
            return carry

        lax.fori_loop(0, NS2, dsuper, 0)
        plsc.subcore_barrier()

        zi = jnp.zeros((16,), I32)
        for q in range(RPT // B2):
            pltpu.sync_copy(den_sh.at[pl.ds(r0 + q * B2, B2)], exr2[0])
            for g in range(B2 // 16):
                rows = lax.iota(I32, 16) + g * 16
                dencol[pl.ds(q * B2 + g * 16, 16)] = plsc.load_gather(
                    exr2[0], [rows, zi])
        pltpu.sync_copy(dencol, den1d_sh.at[pl.ds(r0, RPT)])
        plsc.subcore_barrier()
        pltpu.sync_copy(den1d_sh, dentile)

        base = cid * (EP // 2) + sid * EPT

        def mkblk(tt, b):
            for g in range(BA // 16):
                off = tt * BA + g * 16
                sv = sfe[pl.ds(off, 16)]
                dv = dfe[pl.ds(off, 16)]
                db[b][pl.ds(g * 16, 16)] = dv
                if H == 1:
                    il[b][pl.ds(g * 16, 16)] = sv
                else:
                    il[b][pl.ds(g * 16, 16)] = sv * H + h

        def stage(tt, b):
            mkblk(tt, b)
            pltpu.async_copy(xl_hbm.at[il[b]], xlr[b], gs[b])

        def alpha_scatter(tt, b):
            for g in range(BA // 16):
                ev = efe[pl.ds(tt * BA + g * 16, 16)]
                ex = jnp.exp(ev - K)
                dv = plsc.load_gather(dentile, [db[b][pl.ds(g * 16, 16)]])
                abuf[pl.ds(g * 16, 16)] = ex / (dv + 1e-16)

            def edge(j, carry):
                av = plsc.load_gather(abuf, [jnp.full((16,), j, I32)])
                for k in range(8):
                    xlr[b][j, pl.ds(k * 16, 16)] = xlr[b][j, pl.ds(k * 16, 16)] * av
                return carry

            lax.fori_loop(0, BA, edge, 0)
            pltpu.async_copy(xlr[b], out_sh.at[db[b]], ss[b], add=True)

        def waitg(b):
            pltpu.make_async_copy(xl_hbm.at[pl.ds(0, BA)], xlr[b], gs[b]).wait()

        def waits(b):
            pltpu.make_async_copy(xlr[b], out_sh.at[pl.ds(0, BA)], ss[b]).wait()

        def superblk(q, carry):
            off = base + q * SBE
            pltpu.sync_copy(s_hbm.at[pl.ds(off, SBE)], sfe)
            pltpu.sync_copy(d_hbm.at[pl.ds(off, SBE)], dfe)
            pltpu.sync_copy(e_hbm.at[pl.ds(off, SBE)], efe)
            stage(0, 0)
            for tt in range(SB):
                b = tt % 2
                if tt < SB - 1:
                    if tt >= 1:
                        waits(1 - b)
                    stage(tt + 1, 1 - b)
                waitg(b)
                alpha_scatter(tt, b)
            waits(0)
            waits(1)
            return carry

        lax.fori_loop(0, NSUP, superblk, 0)
        plsc.subcore_barrier()

        for q in range(RPT // BA):
            pltpu.sync_copy(out_sh.at[pl.ds(r0 + q * BA, BA)],
                            part_hbm.at[cid, pl.ds(r0 + q * BA, BA)])

    scratch = dict(
        sfe=pltpu.VMEM((SBE,), I32),
        dfe=pltpu.VMEM((SBE,), I32),
        efe=pltpu.VMEM((SBE,), F32),
        db=[pltpu.VMEM((BA,), I32)] * 2,
        il=[pltpu.VMEM((BA,), I32)] * 2,
        abuf=pltpu.VMEM((BA,), F32),
        eb2=pltpu.VMEM((B2S,), F32),
        db2=pltpu.VMEM((B2S,), I32),
        dbs=[pltpu.VMEM((B2,), I32)] * 2,
        exr2=[pltpu.VMEM((B2, 16), F32)] * 2,
        es=[pltpu.SemaphoreType.DMA] * 2,
        xlr=[pltpu.VMEM((BA, 128), F32)] * 2,
        mbuf=pltpu.VMEM((32, 16), F32),
        dencol=pltpu.VMEM((RPT,), F32),
        dentile=pltpu.VMEM((NP,), F32),
        den_sh=pltpu.VMEM_SHARED((NP, 16), F32),
        den1d_sh=pltpu.VMEM_SHARED((NP,), F32),
        out_sh=pltpu.VMEM_SHARED((NP, 128), F32),
        gs=[pltpu.SemaphoreType.DMA] * 2,
        ss=[pltpu.SemaphoreType.DMA] * 2,
    )
    return pl.kernel(
        body,
        out_type=jax.ShapeDtypeStruct((2, NP, 128), F32),
        mesh=_mesh(),
        scratch_types=scratch,
        compiler_params=pltpu.CompilerParams(needs_layout_passes=False, use_tc_tiling_on_sc=False),
    )


R = 400
GRID = N // R


def _mm2_body(x_ref, wl_ref, wr_ref, ol_ref, or_ref):
    xb = x_ref[...]
    ol_ref[...] = jnp.dot(xb, wl_ref[...], preferred_element_type=F32)
    or_ref[...] = jnp.dot(xb, wr_ref[...], preferred_element_type=F32)


def _mm2(x, wl, wr):
    din, k = wl.shape
    return pl.pallas_call(
        _mm2_body,
        grid=(GRID,),
        in_specs=[pl.BlockSpec((R, din), lambda i: (i, 0)),
                  pl.BlockSpec((din, k), lambda i: (0, 0)),
                  pl.BlockSpec((din, k), lambda i: (0, 0))],
        out_specs=[pl.BlockSpec((R, k), lambda i: (i, 0)),
                   pl.BlockSpec((R, k), lambda i: (i, 0))],
        out_shape=[jax.ShapeDtypeStruct((N, k), F32),
                   jax.ShapeDtypeStruct((N, k), F32)],
    )(x, wl, wr)


def _sumstats_body(p_ref, t_ref, st_ref):
    i = pl.program_id(0)
    t = p_ref[0] + p_ref[1]
    t_ref[...] = t
    st = jnp.stack([jnp.sum(t, axis=0), jnp.sum(t * t, axis=0)])

    @pl.when(i == 0)
    def _():
        st_ref[...] = st

    @pl.when(i > 0)
    def _():
        st_ref[...] = st_ref[...] + st


def _sumstats(parts):
    return pl.pallas_call(
        _sumstats_body,
        grid=(GRID,),
        in_specs=[pl.BlockSpec((2, R, 128), lambda i: (0, i, 0))],
        out_specs=[pl.BlockSpec((R, 128), lambda i: (i, 0)),
                   pl.BlockSpec((2, 128), lambda i: (0, 0))],
        out_shape=[jax.ShapeDtypeStruct((N, 128), F32),
                   jax.ShapeDtypeStruct((2, 128), F32)],
    )(parts)


def _bnscale(st, g, b):
    mu = st[0] / N
    var = st[1] / N - mu * mu
    sc = g / jnp.sqrt(var + 1e-5)
    return sc, b - mu * sc


def _dense1_body(t0, t1, t2, t3, s0, s1, s2, s3, g_ref, b_ref, pw_ref, pb_ref,
                 wl_ref, wr_ref, xp_ref, ol_ref, or_ref):
    ts = (t0, t1, t2, t3)
    ss = (s0, s1, s2, s3)
    acc = jnp.broadcast_to(pb_ref[0], (R, 128))
    for hh in range(4):
        sc, sh = _bnscale(ss[hh][...], g_ref[hh], b_ref[hh])
        xh = jnp.maximum(ts[hh][...] * sc + sh, 0.0)
        acc = acc + jnp.dot(xh, pw_ref[hh], preferred_element_type=F32)
    xp_ref[...] = acc
    ol_ref[...] = jnp.dot(acc, wl_ref[...], preferred_element_type=F32)
    or_ref[...] = jnp.dot(acc, wr_ref[...], preferred_element_type=F32)


def _dense1(ts, sts, g, b, pw, pb, wl, wr):
    return pl.pallas_call(
        _dense1_body,
        grid=(GRID,),
        in_specs=[pl.BlockSpec((R, 128), lambda i: (i, 0))] * 4
        + [pl.BlockSpec((2, 128), lambda i: (0, 0))] * 4
        + [pl.BlockSpec((4, 128), lambda i: (0, 0)),
           pl.BlockSpec((4, 128), lambda i: (0, 0)),
           pl.BlockSpec((4, 128, 128), lambda i: (0, 0, 0)),
           pl.BlockSpec((1, 128), lambda i: (0, 0)),
           pl.BlockSpec((128, 128), lambda i: (0, 0)),
           pl.BlockSpec((128, 128), lambda i: (0, 0))],
        out_specs=[pl.BlockSpec((R, 128), lambda i: (i, 0))] * 3,
        out_shape=[jax.ShapeDtypeStruct((N, 128), F32)] * 3,
    )(*ts, *sts, g, b, pw, pb, wl, wr)


def _densemid_body(t_ref, st_ref, g_ref, b_ref, res_ref, wl_ref, wr_ref,
                   xo_ref, ol_ref, or_ref):
    sc, sh = _bnscale(st_ref[...], g_ref[0], b_ref[0])
    xi = jnp.maximum(t_ref[...] * sc + sh, 0.0) + res_ref[...]
    xo_ref[...] = xi
    ol_ref[...] = jnp.dot(xi, wl_ref[...], preferred_element_type=F32)
    or_ref[...] = jnp.dot(xi, wr_ref[...], preferred_element_type=F32)


def _densemid(t, st, g, b, res, wl, wr):
    return pl.pallas_call(
        _densemid_body,
        grid=(GRID,),
        in_specs=[pl.BlockSpec((R, 128), lambda i: (i, 0)),
                  pl.BlockSpec((2, 128), lambda i: (0, 0)),
                  pl.BlockSpec((1, 128), lambda i: (0, 0)),
                  pl.BlockSpec((1, 128), lambda i: (0, 0)),
                  pl.BlockSpec((R, 128), lambda i: (i, 0)),
                  pl.BlockSpec((128, 128), lambda i: (0, 0)),
                  pl.BlockSpec((128, 128), lambda i: (0, 0))],
        out_specs=[pl.BlockSpec((R, 128), lambda i: (i, 0))] * 3,
        out_shape=[jax.ShapeDtypeStruct((N, 128), F32)] * 3,
    )(t, st, g, b, res, wl, wr)


def _densefinal_body(t_ref, st_ref, g_ref, b_ref, res_ref, bt_ref, lw_ref,
                     lb_ref, o_ref, acc_ref):
    i = pl.program_id(0)
    sc, sh = _bnscale(st_ref[...], g_ref[0], b_ref[0])
    x5 = jnp.maximum(t_ref[...] * sc + sh, 0.0) + res_ref[...]
    bb = bt_ref[0, 0]
    oh = (bb[:, None] == lax.broadcasted_iota(I32, (R, NG), 1)).astype(F32)
    pp = lax.dot_general(oh, x5, (((0,), (0,)), ((), ())),
                         preferred_element_type=F32)

    @pl.when(i == 0)
    def _():
        acc_ref[...] = pp

    @pl.when(i > 0)
    def _():
        acc_ref[...] = acc_ref[...] + pp

    @pl.when(i == GRID - 1)
    def _():
        o_ref[...] = (jnp.dot(acc_ref[...], lw_ref[...],
                              preferred_element_type=F32) + lb_ref[...])


def _densefinal(t, st, g, b, res, batch3d, lw, lb):
    return pl.pallas_call(
        _densefinal_body,
        grid=(GRID,),
        in_specs=[pl.BlockSpec((R, 128), lambda i: (i, 0)),
                  pl.BlockSpec((2, 128), lambda i: (0, 0)),
                  pl.BlockSpec((1, 128), lambda i: (0, 0)),
                  pl.BlockSpec((1, 128), lambda i: (0, 0)),
                  pl.BlockSpec((R, 128), lambda i: (i, 0)),
                  pl.BlockSpec((1, 1, R), lambda i: (i, 0, 0)),
                  pl.BlockSpec((128, NCLS), lambda i: (0, 0)),
                  pl.BlockSpec((1, NCLS), lambda i: (0, 0))],
        out_specs=pl.BlockSpec((NG, NCLS), lambda i: (0, 0)),
        out_shape=jax.ShapeDtypeStruct((NG, NCLS), F32),
        scratch_shapes=[pltpu.VMEM((NG, 128), F32)],
    )(t, st, g, b, res, batch3d, lw, lb)


_make_e1 = functools.cache(_make_e1)
_make_e3 = functools.cache(_make_e3)


def _gat_unit(e1, e3, xlv, xrv, att, s_full, d_full):
    e, mx = e1(xlv, xrv, att, s_full, d_full)
    return e3(xlv, s_full, d_full, e, mx)


def kernel(x, edge_index, batch, gat1_Wl, gat1_Wr, gat1_att, gat1_b, proj1_W,
           proj1_b, gat2_Wl, gat2_Wr, gat2_att, gat2_b, gat3_Wl, gat3_Wr,
           gat3_att, gat3_b, gat4_Wl, gat4_Wr, gat4_att, gat4_b, gat5_Wl,
           gat5_Wr, gat5_att, gat5_b, bn1_g, bn1_b, bn2_g, bn2_b, bn3_g,
           bn3_b, bn4_g, bn4_b, bn5_g, bn5_b, lin_W, lin_b):
    loop = jnp.arange(N, dtype=I32)
    padz = jnp.zeros((EP - ETOT,), I32)
    s_full = jnp.concatenate([edge_index[0].astype(I32), loop, padz])
    d_full = jnp.concatenate([edge_index[1].astype(I32), loop, padz])
    batch3d = batch.astype(I32).reshape(GRID, 1, R)

    xl1, xr1 = _mm2(x, gat1_Wl, gat1_Wr)
    xl1v = xl1.reshape(4 * N, 128)
    xr1v = xr1.reshape(4 * N, 128)
    ts, sts = [], []
    for h in range(4):
        parts = _gat_unit(_make_e1(4, h, 4 * N), _make_e3(4, h, 4 * N),
                          xl1v, xr1v, gat1_att, s_full, d_full)
        t_h, st_h = _sumstats(parts)
        ts.append(t_h)
        sts.append(st_h)
    x1p, xl, xr = _dense1(ts, sts, bn1_g.reshape(4, 128), bn1_b.reshape(4, 128),
                          proj1_W.reshape(4, 128, 128), proj1_b.reshape(1, 128),
                          gat2_Wl, gat2_Wr)

    res = x1p
    mids = [(gat2_att, bn2_g, bn2_b, gat3_Wl, gat3_Wr),
            (gat3_att, bn3_g, bn3_b, gat4_Wl, gat4_Wr),
            (gat4_att, bn4_g, bn4_b, gat5_Wl, gat5_Wr)]
    for att, g, bb, nwl, nwr in mids:
        parts = _gat_unit(_make_e1(1, 0, N), _make_e3(1, 0, N), xl, xr, att,
                          s_full, d_full)
        t, st = _sumstats(parts)
        res, xl, xr = _densemid(t, st, g.reshape(1, 128), bb.reshape(1, 128),
                                res, nwl, nwr)

    parts = _gat_unit(_make_e1(1, 0, N), _make_e3(1, 0, N), xl, xr, gat5_att,
                      s_full, d_full)
    t, st = _sumstats(parts)
    return _densefinal(t, st, bn5_g.reshape(1, 128), bn5_b.reshape(1, 128),
                       res, batch3d, lin_W, lin_b.reshape(1, NCLS))


# speedup vs baseline: 1.0893x; 1.0893x over previous
import functools

import jax
import jax.numpy as jnp
from jax import lax
from jax.experimental import pallas as pl
from jax.experimental.pallas import tpu as pltpu
from jax.experimental.pallas import tpu_sc as plsc

F32 = jnp.float32
I32 = jnp.int32

N = 10000
NP = 10240
NG = 64
NCLS = 7
ETOT = 170000
EP = 172032
EPT = EP // 32
B = 192
NBLK = EPT // B
EPS = EP // 16
B2 = 512
NBLK2 = EPS // B2
RPT = NP // 16
NEG = -1.0e30

@functools.cache
def _mesh():
    return plsc.VectorSubcoreMesh(core_axis_name="c", subcore_axis_name="s",
                                  num_cores=2, num_subcores=16)


def _leaky(z):
    return jnp.where(z > 0, z, z * 0.2)


def _make_e1(H, h, NH):
    def body(xl_hbm, xr_hbm, att_hbm, s_hbm, d_hbm, e_hbm, mx_hbm,
             sfull, dfull, il, ir, xlr, xrr, efull, attv, mxv, gsl, gsr):
        cid = lax.axis_index("c")
        sid = lax.axis_index("s")
        wid = sid * 2 + cid
        base = wid * EPT

        pltpu.sync_copy(att_hbm.at[h], attv)
        attc = [attv[pl.ds(k * 16, 16)] for k in range(8)]
        mxv[...] = jnp.full((16,), NEG, F32)
        pltpu.sync_copy(s_hbm.at[pl.ds(base, EPT)], sfull)
        pltpu.sync_copy(d_hbm.at[pl.ds(base, EPT)], dfull)
        i0 = lax.iota(I32, 16)
        shuf = [(i0 ^ 8).reshape(16, 1), (i0 ^ 4).reshape(16, 1),
                (i0 ^ 2).reshape(16, 1), (i0 ^ 1).reshape(16, 1)]
        _dn = lax.GatherDimensionNumbers(offset_dims=(),
                                         collapsed_slice_dims=(0,),
                                         start_index_map=(0,))

        def _perm(v, ix):
            return lax.gather(v, ix, _dn, (1,),
                              mode=lax.GatherScatterMode.PROMISE_IN_BOUNDS)

        def stage(i, b):
            off = i * B
            for g in range(B // 16):
                sv = sfull[pl.ds(off + g * 16, 16)]
                dv = dfull[pl.ds(off + g * 16, 16)]
                if H == 1:
                    il[b][pl.ds(g * 16, 16)] = sv
                    ir[b][pl.ds(g * 16, 16)] = dv
                else:
                    il[b][pl.ds(g * 16, 16)] = sv * H + h
                    ir[b][pl.ds(g * 16, 16)] = dv * H + h
            pltpu.async_copy(xl_hbm.at[il[b]], xlr[b], gsl[b])
            pltpu.async_copy(xr_hbm.at[ir[b]], xrr[b], gsr[b])

        def waitg(b):
            pltpu.make_async_copy(xl_hbm.at[pl.ds(0, B)], xlr[b], gsl[b]).wait()
            pltpu.make_async_copy(xr_hbm.at[pl.ds(0, B)], xrr[b], gsr[b]).wait()

        def compute(i, b):
            eb = base + i * B
            off = i * B

            def edge(j, carry):
                acc = jnp.zeros((16,), F32)
                for k in range(8):
                    z = xlr[b][j, pl.ds(k * 16, 16)] + xrr[b][j, pl.ds(k * 16, 16)]
                    acc = acc + _leaky(z) * attc[k]
                for ix in shuf:
                    acc = acc + _perm(acc, ix)
                plsc.store_scatter(efull, [jnp.full((16,), off + j, I32)], acc)
                return carry

            lax.fori_loop(0, B, edge, 0)
            for g in range(B // 16):
                ev = efull[pl.ds(off + g * 16, 16)]
                gid = eb + g * 16 + lax.iota(I32, 16)
                ev = jnp.where(gid < ETOT, ev, NEG)
                efull[pl.ds(off + g * 16, 16)] = ev
                mxv[...] = jnp.maximum(mxv[...], ev)

        stage(0, 0)

        def pair(g, carry):
            for b in (0, 1):
                i = g * 2 + b

                @pl.when(i + 1 < NBLK)
                def _():
                    stage(i + 1, 1 - b)

                waitg(b)
                compute(i, b)
            return carry

        lax.fori_loop(0, NBLK // 2, pair, 0)
        pltpu.sync_copy(efull, e_hbm.at[pl.ds(base, EPT)])
        pltpu.sync_copy(mxv, mx_hbm.at[wid])

    scratch = dict(
        sfull=pltpu.VMEM((EPT,), I32),
        dfull=pltpu.VMEM((EPT,), I32),
        il=[pltpu.VMEM((B,), I32)] * 2,
        ir=[pltpu.VMEM((B,), I32)] * 2,
        xlr=[pltpu.VMEM((B, 128), F32)] * 2,
        xrr=[pltpu.VMEM((B, 128), F32)] * 2,
        efull=pltpu.VMEM((EPT,), F32),
        attv=pltpu.VMEM((128,), F32),
        mxv=pltpu.VMEM((16,), F32),
        gsl=[pltpu.SemaphoreType.DMA] * 2,
        gsr=[pltpu.SemaphoreType.DMA] * 2,
    )
    return pl.kernel(
        body,
        out_type=(jax.ShapeDtypeStruct((EP,), F32),
                  jax.ShapeDtypeStruct((32, 16), F32)),
        mesh=_mesh(),
        scratch_types=scratch,
        compiler_params=pltpu.CompilerParams(needs_layout_passes=False, use_tc_tiling_on_sc=False),
    )


BA = 64
NBLKA = EPT // BA
SB = 12
SBE = SB * BA
NSUP = NBLKA // SB
B2 = 64
B2S = 512
NS2 = EPS // B2S


def _make_e3(H, h, NH):
    def body(xl_hbm, s_hbm, d_hbm, e_hbm, mx_hbm, part_hbm,
             sfe, dfe, efe, db, il, abuf, eb2, db2, dbs, exr2, es, xlr, mbuf,
             dencol, dentile, den_sh, den1d_sh, out_sh, gs, ss):
        cid = lax.axis_index("c")
        sid = lax.axis_index("s")
        r0 = sid * RPT

        def zex(j, carry):
            exr2[0][j, pl.ds(0, 16)] = jnp.zeros((16,), F32)
            exr2[1][j, pl.ds(0, 16)] = jnp.zeros((16,), F32)
            return carry

        lax.fori_loop(0, B2, zex, 0)

        def zxl(j, carry):
            for k in range(8):
                xlr[0][j, pl.ds(k * 16, 16)] = jnp.zeros((16,), F32)
            return carry

        lax.fori_loop(0, BA, zxl, 0)
        for q in range(RPT // BA):
            pltpu.sync_copy(xlr[0], out_sh.at[pl.ds(r0 + q * BA, BA)])
        for q in range(RPT // B2):
            pltpu.sync_copy(exr2[0], den_sh.at[pl.ds(r0 + q * B2, B2)])
        plsc.subcore_barrier()

        pltpu.sync_copy(mx_hbm, mbuf)
        mv = mbuf[0, pl.ds(0, 16)]
        for i in range(1, 32):
            mv = jnp.maximum(mv, mbuf[i, pl.ds(0, 16)])
        K = jnp.max(mv)

        def dsuper(q, carry):
            off = sid * EPS + q * B2S
            pltpu.sync_copy(e_hbm.at[pl.ds(off, B2S)], eb2)
            pltpu.sync_copy(d_hbm.at[pl.ds(off, B2S)], db2)
            for tt in range(B2S // B2):
                b = tt % 2
                if tt >= 2:
                    pltpu.make_async_copy(exr2[b], den_sh.at[pl.ds(0, B2)],
                                          es[b]).wait()
                for g in range(B2 // 16):
                    ev = eb2[pl.ds(tt * B2 + g * 16, 16)]
                    ex = jnp.exp(ev - K)
                    rows = lax.iota(I32, 16) + g * 16
                    plsc.store_scatter(exr2[b], [rows, jnp.zeros((16,), I32)],
                                       ex)
                    dbs[b][pl.ds(g * 16, 16)] = db2[pl.ds(tt * B2 + g * 16, 16)]
                pltpu.async_copy(exr2[b], den_sh.at[dbs[b]], es[b], add=True)
            for b in (0, 1):
                pltpu.make_async_copy(exr2[b], den_sh.at[pl.ds(0, B2)],
                                      es[b]).wait()
            return carry

        lax.fori_loop(0, NS2, dsuper, 0)
        plsc.subcore_barrier()

        zi = jnp.zeros((16,), I32)
        for q in range(RPT // B2):
            pltpu.sync_copy(den_sh.at[pl.ds(r0 + q * B2, B2)], exr2[0])
            for g in range(B2 // 16):
                rows = lax.iota(I32, 16) + g * 16
                dencol[pl.ds(q * B2 + g * 16, 16)] = plsc.load_gather(
                    exr2[0], [rows, zi])
        pltpu.sync_copy(dencol, den1d_sh.at[pl.ds(r0, RPT)])
        plsc.subcore_barrier()
        pltpu.sync_copy(den1d_sh, dentile)

        base = cid * (EP // 2) + sid * EPT

        def mkblk(tt, b):
            for g in range(BA // 16):
                off = tt * BA + g * 16
                sv = sfe[pl.ds(off, 16)]
                dv = dfe[pl.ds(off, 16)]
                db[b][pl.ds(g * 16, 16)] = dv
                if H == 1:
                    il[b][pl.ds(g * 16, 16)] = sv
                else:
                    il[b][pl.ds(g * 16, 16)] = sv * H + h

        def stage(tt, b):
            mkblk(tt, b)
            pltpu.async_copy(xl_hbm.at[il[b]], xlr[b], gs[b])

        def alpha_scatter(tt, b):
            for g in range(BA // 16):
                ev = efe[pl.ds(tt * BA + g * 16, 16)]
                ex = jnp.exp(ev - K)
                dv = plsc.load_gather(dentile, [db[b][pl.ds(g * 16, 16)]])
                abuf[pl.ds(g * 16, 16)] = ex / (dv + 1e-16)

            def edge(j, carry):
                av = plsc.load_gather(abuf, [jnp.full((16,), j, I32)])
                for k in range(8):
                    xlr[b][j, pl.ds(k * 16, 16)] = xlr[b][j, pl.ds(k * 16, 16)] * av
                return carry

            lax.fori_loop(0, BA, edge, 0)
            pltpu.async_copy(xlr[b], out_sh.at[db[b]], ss[b], add=True)

        def waitg(b):
            pltpu.make_async_copy(xl_hbm.at[pl.ds(0, BA)], xlr[b], gs[b]).wait()

        def waits(b):
            pltpu.make_async_copy(xlr[b], out_sh.at[pl.ds(0, BA)], ss[b]).wait()

        def superblk(q, carry):
            off = base + q * SBE
            pltpu.sync_copy(s_hbm.at[pl.ds(off, SBE)], sfe)
            pltpu.sync_copy(d_hbm.at[pl.ds(off, SBE)], dfe)
            pltpu.sync_copy(e_hbm.at[pl.ds(off, SBE)], efe)
            stage(0, 0)
            for tt in range(SB):
                b = tt % 2
                if tt < SB - 1:
                    if tt >= 1:
                        waits(1 - b)
                    stage(tt + 1, 1 - b)
                waitg(b)
                alpha_scatter(tt, b)
            waits(0)
            waits(1)
            return carry

        lax.fori_loop(0, NSUP, superblk, 0)
        plsc.subcore_barrier()

        for q in range(RPT // BA):
            pltpu.sync_copy(out_sh.at[pl.ds(r0 + q * BA, BA)],
                            part_hbm.at[cid, pl.ds(r0 + q * BA, BA)])

    scratch = dict(
        sfe=pltpu.VMEM((SBE,), I32),
        dfe=pltpu.VMEM((SBE,), I32),
        efe=pltpu.VMEM((SBE,), F32),
        db=[pltpu.VMEM((BA,), I32)] * 2,
        il=[pltpu.VMEM((BA,), I32)] * 2,
        abuf=pltpu.VMEM((BA,), F32),
        eb2=pltpu.VMEM((B2S,), F32),
        db2=pltpu.VMEM((B2S,), I32),
        dbs=[pltpu.VMEM((B2,), I32)] * 2,
        exr2=[pltpu.VMEM((B2, 16), F32)] * 2,
        es=[pltpu.SemaphoreType.DMA] * 2,
        xlr=[pltpu.VMEM((BA, 128), F32)] * 2,
        mbuf=pltpu.VMEM((32, 16), F32),
        dencol=pltpu.VMEM((RPT,), F32),
        dentile=pltpu.VMEM((NP,), F32),
        den_sh=pltpu.VMEM_SHARED((NP, 16), F32),
        den1d_sh=pltpu.VMEM_SHARED((NP,), F32),
        out_sh=pltpu.VMEM_SHARED((NP, 128), F32),
        gs=[pltpu.SemaphoreType.DMA] * 2,
        ss=[pltpu.SemaphoreType.DMA] * 2,
    )
    return pl.kernel(
        body,
        out_type=jax.ShapeDtypeStruct((2, NP, 128), F32),
        mesh=_mesh(),
        scratch_types=scratch,
        compiler_params=pltpu.CompilerParams(needs_layout_passes=False, use_tc_tiling_on_sc=False),
    )


R = 400
GRID = N // R


def _mm2_body(x_ref, wl_ref, wr_ref, ol_ref, or_ref):
    xb = x_ref[...]
    ol_ref[...] = jnp.dot(xb, wl_ref[...], preferred_element_type=F32)
    or_ref[...] = jnp.dot(xb, wr_ref[...], preferred_element_type=F32)


def _mm2(x, wl, wr):
    din, k = wl.shape
    return pl.pallas_call(
        _mm2_body,
        grid=(GRID,),
        in_specs=[pl.BlockSpec((R, din), lambda i: (i, 0)),
                  pl.BlockSpec((din, k), lambda i: (0, 0)),
                  pl.BlockSpec((din, k), lambda i: (0, 0))],
        out_specs=[pl.BlockSpec((R, k), lambda i: (i, 0)),
                   pl.BlockSpec((R, k), lambda i: (i, 0))],
        out_shape=[jax.ShapeDtypeStruct((N, k), F32),
                   jax.ShapeDtypeStruct((N, k), F32)],
    )(x, wl, wr)


def _sumstats_body(p_ref, t_ref, st_ref):
    i = pl.program_id(0)
    t = p_ref[0] + p_ref[1]
    t_ref[...] = t
    st = jnp.stack([jnp.sum(t, axis=0), jnp.sum(t * t, axis=0)])

    @pl.when(i == 0)
    def _():
        st_ref[...] = st

    @pl.when(i > 0)
    def _():
        st_ref[...] = st_ref[...] + st


def _sumstats(parts):
    return pl.pallas_call(
        _sumstats_body,
        grid=(GRID,),
        in_specs=[pl.BlockSpec((2, R, 128), lambda i: (0, i, 0))],
        out_specs=[pl.BlockSpec((R, 128), lambda i: (i, 0)),
                   pl.BlockSpec((2, 128), lambda i: (0, 0))],
        out_shape=[jax.ShapeDtypeStruct((N, 128), F32),
                   jax.ShapeDtypeStruct((2, 128), F32)],
    )(parts)


def _bnscale(st, g, b):
    mu = st[0] / N
    var = st[1] / N - mu * mu
    sc = g / jnp.sqrt(var + 1e-5)
    return sc, b - mu * sc


def _dense1_body(t0, t1, t2, t3, s0, s1, s2, s3, g_ref, b_ref, pw_ref, pb_ref,
                 wl_ref, wr_ref, xp_ref, ol_ref, or_ref):
    ts = (t0, t1, t2, t3)
    ss = (s0, s1, s2, s3)
    acc = jnp.broadcast_to(pb_ref[0], (R, 128))
    for hh in range(4):
        sc, sh = _bnscale(ss[hh][...], g_ref[hh], b_ref[hh])
        xh = jnp.maximum(ts[hh][...] * sc + sh, 0.0)
        acc = acc + jnp.dot(xh, pw_ref[hh], preferred_element_type=F32)
    xp_ref[...] = acc
    ol_ref[...] = jnp.dot(acc, wl_ref[...], preferred_element_type=F32)
    or_ref[...] = jnp.dot(acc, wr_ref[...], preferred_element_type=F32)


def _dense1(ts, sts, g, b, pw, pb, wl, wr):
    return pl.pallas_call(
        _dense1_body,
        grid=(GRID,),
        in_specs=[pl.BlockSpec((R, 128), lambda i: (i, 0))] * 4
        + [pl.BlockSpec((2, 128), lambda i: (0, 0))] * 4
        + [pl.BlockSpec((4, 128), lambda i: (0, 0)),
           pl.BlockSpec((4, 128), lambda i: (0, 0)),
           pl.BlockSpec((4, 128, 128), lambda i: (0, 0, 0)),
           pl.BlockSpec((1, 128), lambda i: (0, 0)),
           pl.BlockSpec((128, 128), lambda i: (0, 0)),
           pl.BlockSpec((128, 128), lambda i: (0, 0))],
        out_specs=[pl.BlockSpec((R, 128), lambda i: (i, 0))] * 3,
        out_shape=[jax.ShapeDtypeStruct((N, 128), F32)] * 3,
    )(*ts, *sts, g, b, pw, pb, wl, wr)


def _densemid_body(t_ref, st_ref, g_ref, b_ref, res_ref, wl_ref, wr_ref,
                   xo_ref, ol_ref, or_ref):
    sc, sh = _bnscale(st_ref[...], g_ref[0], b_ref[0])
    xi = jnp.maximum(t_ref[...] * sc + sh, 0.0) + res_ref[...]
    xo_ref[...] = xi
    ol_ref[...] = jnp.dot(xi, wl_ref[...], preferred_element_type=F32)
    or_ref[...] = jnp.dot(xi, wr_ref[...], preferred_element_type=F32)


def _densemid(t, st, g, b, res, wl, wr):
    return pl.pallas_call(
        _densemid_body,
        grid=(GRID,),
        in_specs=[pl.BlockSpec((R, 128), lambda i: (i, 0)),
                  pl.BlockSpec((2, 128), lambda i: (0, 0)),
                  pl.BlockSpec((1, 128), lambda i: (0, 0)),
                  pl.BlockSpec((1, 128), lambda i: (0, 0)),
                  pl.BlockSpec((R, 128), lambda i: (i, 0)),
                  pl.BlockSpec((128, 128), lambda i: (0, 0)),
                  pl.BlockSpec((128, 128), lambda i: (0, 0))],
        out_specs=[pl.BlockSpec((R, 128), lambda i: (i, 0))] * 3,
        out_shape=[jax.ShapeDtypeStruct((N, 128), F32)] * 3,
    )(t, st, g, b, res, wl, wr)


def _densefinal_body(t_ref, st_ref, g_ref, b_ref, res_ref, bt_ref, lw_ref,
                     lb_ref, o_ref, acc_ref):
    i = pl.program_id(0)
    sc, sh = _bnscale(st_ref[...], g_ref[0], b_ref[0])
    x5 = jnp.maximum(t_ref[...] * sc + sh, 0.0) + res_ref[...]
    bb = bt_ref[0, 0]
    oh = (bb[:, None] == lax.broadcasted_iota(I32, (R, NG), 1)).astype(F32)
    pp = lax.dot_general(oh, x5, (((0,), (0,)), ((), ())),
                         preferred_element_type=F32)

    @pl.when(i == 0)
    def _():
        acc_ref[...] = pp

    @pl.when(i > 0)
    def _():
        acc_ref[...] = acc_ref[...] + pp

    @pl.when(i == GRID - 1)
    def _():
        o_ref[...] = (jnp.dot(acc_ref[...], lw_ref[...],
                              preferred_element_type=F32) + lb_ref[...])


def _densefinal(t, st, g, b, res, batch3d, lw, lb):
    return pl.pallas_call(
        _densefinal_body,
        grid=(GRID,),
        in_specs=[pl.BlockSpec((R, 128), lambda i: (i, 0)),
                  pl.BlockSpec((2, 128), lambda i: (0, 0)),
                  pl.BlockSpec((1, 128), lambda i: (0, 0)),
                  pl.BlockSpec((1, 128), lambda i: (0, 0)),
                  pl.BlockSpec((R, 128), lambda i: (i, 0)),
                  pl.BlockSpec((1, 1, R), lambda i: (i, 0, 0)),
                  pl.BlockSpec((128, NCLS), lambda i: (0, 0)),
                  pl.BlockSpec((1, NCLS), lambda i: (0, 0))],
        out_specs=pl.BlockSpec((NG, NCLS), lambda i: (0, 0)),
        out_shape=jax.ShapeDtypeStruct((NG, NCLS), F32),
        scratch_shapes=[pltpu.VMEM((NG, 128), F32)],
    )(t, st, g, b, res, batch3d, lw, lb)


_make_e1 = functools.cache(_make_e1)
_make_e3 = functools.cache(_make_e3)


def _gat_unit(e1, e3, xlv, xrv, att, s_full, d_full):
    e, mx = e1(xlv, xrv, att, s_full, d_full)
    return e3(xlv, s_full, d_full, e, mx)


def kernel(x, edge_index, batch, gat1_Wl, gat1_Wr, gat1_att, gat1_b, proj1_W,
           proj1_b, gat2_Wl, gat2_Wr, gat2_att, gat2_b, gat3_Wl, gat3_Wr,
           gat3_att, gat3_b, gat4_Wl, gat4_Wr, gat4_att, gat4_b, gat5_Wl,
           gat5_Wr, gat5_att, gat5_b, bn1_g, bn1_b, bn2_g, bn2_b, bn3_g,
           bn3_b, bn4_g, bn4_b, bn5_g, bn5_b, lin_W, lin_b):
    loop = jnp.arange(N, dtype=I32)
    padz = jnp.zeros((EP - ETOT,), I32)
    s_full = jnp.concatenate([edge_index[0].astype(I32), loop, padz])
    d_full = jnp.concatenate([edge_index[1].astype(I32), loop, padz])
    batch3d = batch.astype(I32).reshape(GRID, 1, R)

    xl1, xr1 = _mm2(x, gat1_Wl, gat1_Wr)
    xl1v = xl1.reshape(4 * N, 128)
    xr1v = xr1.reshape(4 * N, 128)
    ts, sts = [], []
    for h in range(4):
        parts = _gat_unit(_make_e1(4, h, 4 * N), _make_e3(4, h, 4 * N),
                          xl1v, xr1v, gat1_att, s_full, d_full)
        t_h, st_h = _sumstats(parts)
        ts.append(t_h)
        sts.append(st_h)
    x1p, xl, xr = _dense1(ts, sts, bn1_g.reshape(4, 128), bn1_b.reshape(4, 128),
                          proj1_W.reshape(4, 128, 128), proj1_b.reshape(1, 128),
                          gat2_Wl, gat2_Wr)

    res = x1p
    mids = [(gat2_att, bn2_g, bn2_b, gat3_Wl, gat3_Wr),
            (gat3_att, bn3_g, bn3_b, gat4_Wl, gat4_Wr),
            (gat4_att, bn4_g, bn4_b, gat5_Wl, gat5_Wr)]
    for att, g, bb, nwl, nwr in mids:
        parts = _gat_unit(_make_e1(1, 0, N), _make_e3(1, 0, N), xl, xr, att,
                          s_full, d_full)
        t, st = _sumstats(parts)
        res, xl, xr = _densemid(t, st, g.reshape(1, 128), bb.reshape(1, 128),
                                res, nwl, nwr)

    parts = _gat_unit(_make_e1(1, 0, N), _make_e3(1, 0, N), xl, xr, gat5_att,
                      s_full, d_full)
    t, st = _sumstats(parts)
    return _densefinal(t, st, bn5_g.reshape(1, 128), bn5_b.reshape(1, 128),
                       res, batch3d, lin_W, lin_b.reshape(1, NCLS))
